# baseline (speedup 1.0000x reference)
import functools

import jax
import jax.numpy as jnp
from jax import lax
from jax.experimental import pallas as pl
from jax.experimental.pallas import tpu as pltpu
from jax.experimental.pallas import tpu_sc as plsc

N_NODES = 10000
N_EDGES = 320000
D = 128

NCORES = 2
NSUB = 16
NWORK = NCORES * NSUB
LANES = 16
EPW = N_EDGES // NWORK
CHUNK = 80
NCHUNK = EPW // CHUNK
GROUPS = CHUNK // LANES
EUNROLL = 8


def _pack_pair(lo_bf16, hi_bf16):
    lo = jax.lax.bitcast_convert_type(lo_bf16, jnp.uint16).astype(jnp.uint32)
    hi = jax.lax.bitcast_convert_type(hi_bf16, jnp.uint16).astype(jnp.uint32)
    return jax.lax.bitcast_convert_type((hi << 16) | lo, jnp.int32)


def _tc_transform_body(circ_ref, dis_ref, w_ref, wc_ref, t_out, d_out):
    w0 = w_ref[0]
    w1 = w_ref[1]
    m0 = w0 * wc_ref[0, 0] + w1 * wc_ref[1, 0]
    m1 = w0 * wc_ref[0, 1] + w1 * wc_ref[1, 1]
    c = circ_ref[...]
    t0 = jnp.dot(c, m0, preferred_element_type=jnp.float32
                 ).astype(jnp.bfloat16)
    t1 = jnp.dot(c, m1, preferred_element_type=jnp.float32
                 ).astype(jnp.bfloat16)
    t_out[...] = _pack_pair(t0, t1)
    d = dis_ref[...].astype(jnp.bfloat16)
    d_out[...] = _pack_pair(d[:, :D // 2], d[:, D // 2:])


def _tc_transform(circ, dis, weight, wc):
    return pl.pallas_call(
        _tc_transform_body,
        out_shape=(
            jax.ShapeDtypeStruct((N_NODES, D), jnp.int32),
            jax.ShapeDtypeStruct((N_NODES, D // 2), jnp.int32),
        ),
        in_specs=[
            pl.BlockSpec(memory_space=pltpu.VMEM),
            pl.BlockSpec(memory_space=pltpu.VMEM),
            pl.BlockSpec(memory_space=pltpu.VMEM),
            pl.BlockSpec(memory_space=pltpu.SMEM),
        ],
        out_specs=(pl.BlockSpec(memory_space=pltpu.VMEM),
                   pl.BlockSpec(memory_space=pltpu.VMEM)),
    )(circ, dis, weight, wc)


def _sc_edge_body(t_hbm, dis_hbm, ci_hbm, di_hbm, o0_hbm, o1_hbm,
                  cidx_v, didx_v, rows_t0, rows_d0, rows_t1, rows_d1,
                  rows_t2, rows_d2, o0_v, o1_v, sem0, sem1, sem2):
    wid = lax.axis_index("s") * NCORES + lax.axis_index("c")
    base = wid * EPW

    pltpu.sync_copy(ci_hbm.at[pl.ds(base, EPW)], cidx_v)
    pltpu.sync_copy(di_hbm.at[pl.ds(base, EPW)], didx_v)

    lane15 = lax.iota(jnp.int32, LANES) == (LANES - 1)

    def copies(c, rt, rd, sem):
        off = pl.multiple_of(c * CHUNK, CHUNK)
        return (
            pltpu.make_async_copy(
                t_hbm.at[cidx_v.at[pl.ds(off, CHUNK)]], rt, sem),
            pltpu.make_async_copy(
                dis_hbm.at[didx_v.at[pl.ds(off, CHUNK)]], rd, sem),
        )

    def gather_start(c, rt, rd, sem):
        cp_t, cp_d = copies(c, rt, rd, sem)
        cp_t.start()
        cp_d.start()

    def gather_wait(c, rt, rd, sem):
        cp_t, cp_d = copies(c, rt, rd, sem)
        cp_t.wait()
        cp_d.wait()

    def compute(c, rt, rd):
        off = pl.multiple_of(c * CHUNK, CHUNK)

        @plsc.parallel_loop(0, CHUNK, step=1, unroll=EUNROLL)
        def _(e):
            a0 = jnp.zeros((LANES,), jnp.float32)
            a1 = jnp.zeros((LANES,), jnp.float32)
            ilv = plsc.PackFormat.INTERLEAVED
            for q in range(D // (2 * LANES)):
                dp = plsc.bitcast(rd[e, pl.ds(q * LANES, LANES)],
                                  jnp.bfloat16)
                da, db = plsc.unpack(dp, format=ilv)
                ta = plsc.bitcast(rt[e, pl.ds(q * LANES, LANES)],
                                  jnp.bfloat16)
                t0a, t1a = plsc.unpack(ta, format=ilv)
                tb = plsc.bitcast(rt[e, pl.ds(D // 2 + q * LANES, LANES)],
                                  jnp.bfloat16)
                t0b, t1b = plsc.unpack(tb, format=ilv)
                a0 = a0 + t0a * da + t0b * db
                a1 = a1 + t1a * da + t1b * db
            c0 = jnp.maximum(plsc.cumsum(a0), 0.0)
            c1 = jnp.maximum(plsc.cumsum(a1), 0.0)
            ids = jnp.full((LANES,), off + e, jnp.int32)
            plsc.store_scatter(o0_v, [ids], c0, mask=lane15)
            plsc.store_scatter(o1_v, [ids], c1, mask=lane15)

    bufs = ((rows_t0, rows_d0, sem0),
            (rows_t1, rows_d1, sem1),
            (rows_t2, rows_d2, sem2))
    for b in range(3):
        gather_start(b, *bufs[b])

    def tri_body(i, carry):
        c = i * 3
        for b in range(3):
            rt, rd, sem = bufs[b]
            gather_wait(c + b, rt, rd, sem)
            compute(c + b, rt, rd)
            nxt = c + b + 3

            @pl.when(nxt < NCHUNK)
            def _():
                gather_start(nxt, rt, rd, sem)
        return carry

    lax.fori_loop(0, NCHUNK // 3, tri_body, 0)
    for c in (NCHUNK - 2, NCHUNK - 1):
        rt, rd, sem = bufs[c % 3]
        gather_wait(c, rt, rd, sem)
        compute(c, rt, rd)

    pltpu.sync_copy(o0_v, o0_hbm.at[pl.ds(base, EPW)])
    pltpu.sync_copy(o1_v, o1_hbm.at[pl.ds(base, EPW)])


@functools.lru_cache(maxsize=1)
def _sc_edge():
  return pl.kernel(
    _sc_edge_body,
    out_type=(
        jax.ShapeDtypeStruct((N_EDGES,), jnp.float32),
        jax.ShapeDtypeStruct((N_EDGES,), jnp.float32),
    ),
    mesh=plsc.VectorSubcoreMesh(core_axis_name="c", subcore_axis_name="s",
                                num_cores=NCORES, num_subcores=NSUB),
    compiler_params=pltpu.CompilerParams(needs_layout_passes=False,
                                         use_tc_tiling_on_sc=False),
    scratch_types=[
        pltpu.VMEM((EPW,), jnp.int32),
        pltpu.VMEM((EPW,), jnp.int32),
        pltpu.VMEM((CHUNK, D), jnp.int32),
        pltpu.VMEM((CHUNK, D // 2), jnp.int32),
        pltpu.VMEM((CHUNK, D), jnp.int32),
        pltpu.VMEM((CHUNK, D // 2), jnp.int32),
        pltpu.VMEM((CHUNK, D), jnp.int32),
        pltpu.VMEM((CHUNK, D // 2), jnp.int32),
        pltpu.VMEM((EPW,), jnp.float32),
        pltpu.VMEM((EPW,), jnp.float32),
        pltpu.SemaphoreType.DMA,
        pltpu.SemaphoreType.DMA,
        pltpu.SemaphoreType.DMA,
    ],
  )


@jax.jit
def kernel(circ_inputs, dis_inputs, weight, weight_classifier,
           circ_indices, dis_indices):
    t, dp = _tc_transform(circ_inputs, dis_inputs, weight, weight_classifier)
    o0, o1 = _sc_edge()(t, dp,
                        circ_indices.astype(jnp.int32),
                        dis_indices.astype(jnp.int32))
    return jnp.stack([o0, o1], axis=1)


# speedup vs baseline: 1.0400x; 1.0400x over previous
import functools

import jax
import jax.numpy as jnp
from jax import lax
from jax.experimental import pallas as pl
from jax.experimental.pallas import tpu as pltpu
from jax.experimental.pallas import tpu_sc as plsc

N_NODES = 10000
N_EDGES = 320000
D = 128

NCORES = 2
NSUB = 16
NWORK = NCORES * NSUB
LANES = 16
EPW = N_EDGES // NWORK
CHUNK = 80
NCHUNK = EPW // CHUNK
GROUPS = CHUNK // LANES
EUNROLL = 8


def _pack_pair(lo_bf16, hi_bf16):
    lo = jax.lax.bitcast_convert_type(lo_bf16, jnp.uint16).astype(jnp.uint32)
    hi = jax.lax.bitcast_convert_type(hi_bf16, jnp.uint16).astype(jnp.uint32)
    return jax.lax.bitcast_convert_type((hi << 16) | lo, jnp.int32)


def _tc_transform_body(circ_ref, dis_ref, w_ref, wc_ref, t_out, d_out):
    w0 = w_ref[0]
    w1 = w_ref[1]
    m0 = w0 * wc_ref[0, 0] + w1 * wc_ref[1, 0]
    m1 = w0 * wc_ref[0, 1] + w1 * wc_ref[1, 1]
    c = circ_ref[...]
    t0 = jnp.dot(c, m0, preferred_element_type=jnp.float32
                 ).astype(jnp.bfloat16)
    t1 = jnp.dot(c, m1, preferred_element_type=jnp.float32
                 ).astype(jnp.bfloat16)
    t_out[...] = _pack_pair(t0, t1)
    d = dis_ref[...].astype(jnp.bfloat16)
    d_out[...] = _pack_pair(d[:, :D // 2], d[:, D // 2:])


def _tc_transform(circ, dis, weight, wc):
    return pl.pallas_call(
        _tc_transform_body,
        out_shape=(
            jax.ShapeDtypeStruct((N_NODES, D), jnp.int32),
            jax.ShapeDtypeStruct((N_NODES, D // 2), jnp.int32),
        ),
        in_specs=[
            pl.BlockSpec(memory_space=pltpu.VMEM),
            pl.BlockSpec(memory_space=pltpu.VMEM),
            pl.BlockSpec(memory_space=pltpu.VMEM),
            pl.BlockSpec(memory_space=pltpu.SMEM),
        ],
        out_specs=(pl.BlockSpec(memory_space=pltpu.VMEM),
                   pl.BlockSpec(memory_space=pltpu.VMEM)),
    )(circ, dis, weight, wc)


def _sc_edge_body(t_hbm, dis_hbm, ci_hbm, di_hbm, o0_hbm, o1_hbm,
                  cidx_v, didx_v, rows_t0, rows_d0, rows_t1, rows_d1,
                  o0_v, o1_v, sem0, sem1):
    wid = lax.axis_index("s") * NCORES + lax.axis_index("c")
    base = wid * EPW

    pltpu.sync_copy(ci_hbm.at[pl.ds(base, EPW)], cidx_v)
    pltpu.sync_copy(di_hbm.at[pl.ds(base, EPW)], didx_v)

    lane15 = lax.iota(jnp.int32, LANES) == (LANES - 1)

    def copies(c, rt, rd, sem):
        off = pl.multiple_of(c * CHUNK, CHUNK)
        return (
            pltpu.make_async_copy(
                t_hbm.at[cidx_v.at[pl.ds(off, CHUNK)]], rt, sem),
            pltpu.make_async_copy(
                dis_hbm.at[didx_v.at[pl.ds(off, CHUNK)]], rd, sem),
        )

    def gather_start(c, rt, rd, sem):
        cp_t, cp_d = copies(c, rt, rd, sem)
        cp_t.start()
        cp_d.start()

    def gather_wait(c, rt, rd, sem):
        cp_t, cp_d = copies(c, rt, rd, sem)
        cp_t.wait()
        cp_d.wait()

    def compute(c, rt, rd):
        off = pl.multiple_of(c * CHUNK, CHUNK)

        @plsc.parallel_loop(0, CHUNK, step=1, unroll=EUNROLL)
        def _(e):
            a0 = jnp.zeros((LANES,), jnp.float32)
            a1 = jnp.zeros((LANES,), jnp.float32)
            ilv = plsc.PackFormat.INTERLEAVED
            for q in range(D // (2 * LANES)):
                dp = plsc.bitcast(rd[e, pl.ds(q * LANES, LANES)],
                                  jnp.bfloat16)
                da, db = plsc.unpack(dp, format=ilv)
                ta = plsc.bitcast(rt[e, pl.ds(q * LANES, LANES)],
                                  jnp.bfloat16)
                t0a, t1a = plsc.unpack(ta, format=ilv)
                tb = plsc.bitcast(rt[e, pl.ds(D // 2 + q * LANES, LANES)],
                                  jnp.bfloat16)
                t0b, t1b = plsc.unpack(tb, format=ilv)
                a0 = a0 + t0a * da + t0b * db
                a1 = a1 + t1a * da + t1b * db
            c0 = jnp.maximum(plsc.cumsum(a0), 0.0)
            c1 = jnp.maximum(plsc.cumsum(a1), 0.0)
            ids = jnp.full((LANES,), off + e, jnp.int32)
            plsc.store_scatter(o0_v, [ids], c0, mask=lane15)
            plsc.store_scatter(o1_v, [ids], c1, mask=lane15)

    gather_start(0, rows_t0, rows_d0, sem0)

    def pair_body(i, carry):
        c0 = i * 2
        c1 = c0 + 1
        gather_wait(c0, rows_t0, rows_d0, sem0)
        gather_start(c1, rows_t1, rows_d1, sem1)
        compute(c0, rows_t0, rows_d0)
        gather_wait(c1, rows_t1, rows_d1, sem1)
        gather_start(c1 + 1, rows_t0, rows_d0, sem0)
        compute(c1, rows_t1, rows_d1)
        return carry

    lax.fori_loop(0, NCHUNK // 2, pair_body, 0)
    gather_wait(NCHUNK - 1, rows_t0, rows_d0, sem0)
    compute(NCHUNK - 1, rows_t0, rows_d0)

    pltpu.sync_copy(o0_v, o0_hbm.at[pl.ds(base, EPW)])
    pltpu.sync_copy(o1_v, o1_hbm.at[pl.ds(base, EPW)])


@functools.lru_cache(maxsize=1)
def _sc_edge():
  return pl.kernel(
    _sc_edge_body,
    out_type=(
        jax.ShapeDtypeStruct((N_EDGES,), jnp.float32),
        jax.ShapeDtypeStruct((N_EDGES,), jnp.float32),
    ),
    mesh=plsc.VectorSubcoreMesh(core_axis_name="c", subcore_axis_name="s",
                                num_cores=NCORES, num_subcores=NSUB),
    compiler_params=pltpu.CompilerParams(needs_layout_passes=False,
                                         use_tc_tiling_on_sc=False),
    scratch_types=[
        pltpu.VMEM((EPW,), jnp.int32),
        pltpu.VMEM((EPW,), jnp.int32),
        pltpu.VMEM((CHUNK, D), jnp.int32),
        pltpu.VMEM((CHUNK, D // 2), jnp.int32),
        pltpu.VMEM((CHUNK, D), jnp.int32),
        pltpu.VMEM((CHUNK, D // 2), jnp.int32),
        pltpu.VMEM((EPW,), jnp.float32),
        pltpu.VMEM((EPW,), jnp.float32),
        pltpu.SemaphoreType.DMA,
        pltpu.SemaphoreType.DMA,
    ],
  )


@jax.jit
def kernel(circ_inputs, dis_inputs, weight, weight_classifier,
           circ_indices, dis_indices):
    t, dp = _tc_transform(circ_inputs, dis_inputs, weight, weight_classifier)
    o0, o1 = _sc_edge()(t, dp,
                        circ_indices.astype(jnp.int32),
                        dis_indices.astype(jnp.int32))
    return jnp.stack([o0, o1], axis=1)
